# Initial kernel scaffold; baseline (speedup 1.0000x reference)
import jax
import jax.numpy as jnp
from jax.experimental import pallas as pl


def kernel(x, Wg, bg, W0, b0, W1, b1, W2, b2):
    raise NotImplementedError("write your pallas kernel here")


# speedup vs baseline: 2.0137x; 2.0137x over previous
import functools

import jax
import jax.numpy as jnp
from jax import lax
from jax.experimental import pallas as pl
from jax.experimental.pallas import tpu as pltpu
from jax.experimental.pallas import tpu_sc as plsc

N = 8192
D = 1024
E = 8
LANES = 128
BN = 512
T = 256
NT = N // T + E
PADDED_N = NT * T

NW = 32
RPW = N // NW
C = 64

f32 = jnp.float32
i32 = jnp.int32


def _route_body(x_ref, wg_ref, bg_ref, idx_ref, rank_ref, counts_ref):
    i = pl.program_id(0)
    logits = jnp.dot(x_ref[...].astype(jnp.bfloat16), wg_ref[...],
                     preferred_element_type=f32) + bg_ref[...]
    m = jnp.max(logits, axis=1, keepdims=True)
    lane = lax.broadcasted_iota(i32, logits.shape, 1)
    idxv = jnp.min(jnp.where(logits == m, lane, LANES - 1), axis=1, keepdims=True)
    oh = (lane == idxv).astype(jnp.bfloat16)
    r = lax.broadcasted_iota(i32, (BN, BN), 0)
    c = lax.broadcasted_iota(i32, (BN, BN), 1)
    ltri = (r > c).astype(jnp.bfloat16)
    csum = jnp.dot(ltri, oh, preferred_element_type=f32)
    oh = oh.astype(f32)
    carry = jnp.where(i == 0, jnp.zeros((1, LANES), f32), counts_ref[...])
    rank = jnp.sum(oh * (csum + carry), axis=1, keepdims=True)
    idx_ref[...] = idxv
    rank_ref[...] = rank.astype(i32)
    counts_ref[...] = carry + jnp.sum(oh, axis=0, keepdims=True)


def _route(x, wg_p, bg_p):
    return pl.pallas_call(
        _route_body,
        grid=(N // BN,),
        in_specs=[
            pl.BlockSpec((BN, D), lambda i: (i, 0)),
            pl.BlockSpec((D, LANES), lambda i: (0, 0)),
            pl.BlockSpec((1, LANES), lambda i: (0, 0)),
        ],
        out_specs=[
            pl.BlockSpec((BN, 1), lambda i: (i, 0)),
            pl.BlockSpec((BN, 1), lambda i: (i, 0)),
            pl.BlockSpec((1, LANES), lambda i: (0, 0)),
        ],
        out_shape=[
            jax.ShapeDtypeStruct((N, 1), i32),
            jax.ShapeDtypeStruct((N, 1), i32),
            jax.ShapeDtypeStruct((1, LANES), f32),
        ],
    )(x, wg_p, bg_p)


def _slot_body(counts_ref, idx_ref, rank_ref, slot_ref, te_ref):
    i = pl.program_id(0)
    starts, ends = [], []
    cum = i32(0)
    for e in range(E):
        cnt = counts_ref[0, e].astype(i32)
        padded = ((cnt + (T - 1)) // T) * T
        starts.append(cum)
        cum = cum + padded
        ends.append(cum)
    idxb = idx_ref[...]
    slot = rank_ref[...]
    for e in range(E):
        slot = slot + jnp.where(idxb == e, starts[e], 0)
    slot_ref[...] = slot

    @pl.when(i == 0)
    def _():
        t = lax.broadcasted_iota(i32, (1, LANES), 1) * T
        acc = jnp.zeros((1, LANES), i32)
        for e in range(E):
            acc = acc + (t >= ends[e]).astype(i32)
        te_ref[...] = jnp.minimum(acc, E - 1)


def _slots(counts, idx2, rank2):
    return pl.pallas_call(
        _slot_body,
        grid=(N // BN,),
        in_specs=[
            pl.BlockSpec(memory_space=pltpu.SMEM),
            pl.BlockSpec((BN, 1), lambda i: (i, 0)),
            pl.BlockSpec((BN, 1), lambda i: (i, 0)),
        ],
        out_specs=[
            pl.BlockSpec((BN, 1), lambda i: (i, 0)),
            pl.BlockSpec((1, LANES), lambda i: (0, 0)),
        ],
        out_shape=[
            jax.ShapeDtypeStruct((N, 1), i32),
            jax.ShapeDtypeStruct((1, LANES), i32),
        ],
    )(counts, idx2, rank2)


def _sc_mesh():
    return plsc.VectorSubcoreMesh(core_axis_name="c", subcore_axis_name="s",
                                  num_cores=2, num_subcores=16)


def _permute(x, slot):
    @functools.partial(
        pl.kernel,
        out_type=jax.ShapeDtypeStruct((PADDED_N, D), f32),
        mesh=_sc_mesh(),
        scratch_types=[
            pltpu.VMEM((C,), i32),
            pltpu.VMEM((C, D), f32),
            pltpu.SemaphoreType.DMA,
        ],
    )
    def body(x_hbm, slot_hbm, xs_hbm, idx_v, rows_v, sem):
        wid = lax.axis_index("s") * 2 + lax.axis_index("c")
        base = wid * RPW
        for j in range(RPW // C):
            off = base + j * C
            pltpu.sync_copy(slot_hbm.at[pl.ds(off, C)], idx_v)
            pltpu.sync_copy(x_hbm.at[pl.ds(off, C)], rows_v)
            pltpu.async_copy(rows_v, xs_hbm.at[idx_v], sem).wait()

    return body(x, slot)


def _unpermute(ys, slot):
    @functools.partial(
        pl.kernel,
        out_type=jax.ShapeDtypeStruct((N, D), f32),
        mesh=_sc_mesh(),
        scratch_types=[
            pltpu.VMEM((C,), i32),
            pltpu.VMEM((C, D), f32),
            pltpu.SemaphoreType.DMA,
        ],
    )
    def body(ys_hbm, slot_hbm, out_hbm, idx_v, rows_v, sem):
        wid = lax.axis_index("s") * 2 + lax.axis_index("c")
        base = wid * RPW
        for j in range(RPW // C):
            off = base + j * C
            pltpu.sync_copy(slot_hbm.at[pl.ds(off, C)], idx_v)
            pltpu.async_copy(ys_hbm.at[idx_v], rows_v, sem).wait()
            pltpu.sync_copy(rows_v, out_hbm.at[pl.ds(off, C)])

    return body(ys, slot)


def _moe_body(te_ref, x_ref, w0_ref, b0_ref, w1_ref, b1_ref, w2_ref, b2_ref, y_ref):
    h = jnp.dot(x_ref[...], w0_ref[0], preferred_element_type=f32)
    h = jnp.maximum(h + b0_ref[0], 0.0).astype(jnp.bfloat16)
    h = jnp.dot(h, w1_ref[0], preferred_element_type=f32)
    h = jnp.maximum(h + b1_ref[0], 0.0).astype(jnp.bfloat16)
    y_ref[...] = jnp.dot(h, w2_ref[0], preferred_element_type=f32) + b2_ref[0]


def _moe(te, xs, W0, b0, W1, b1, W2, b2):
    grid_spec = pltpu.PrefetchScalarGridSpec(
        num_scalar_prefetch=1,
        grid=(NT,),
        in_specs=[
            pl.BlockSpec((T, D), lambda i, te: (i, 0)),
            pl.BlockSpec((1, D, D), lambda i, te: (te[i], 0, 0)),
            pl.BlockSpec((1, 1, D), lambda i, te: (te[i], 0, 0)),
            pl.BlockSpec((1, D, D), lambda i, te: (te[i], 0, 0)),
            pl.BlockSpec((1, 1, D), lambda i, te: (te[i], 0, 0)),
            pl.BlockSpec((1, D, D), lambda i, te: (te[i], 0, 0)),
            pl.BlockSpec((1, 1, D), lambda i, te: (te[i], 0, 0)),
        ],
        out_specs=pl.BlockSpec((T, D), lambda i, te: (i, 0)),
    )
    return pl.pallas_call(
        _moe_body,
        grid_spec=grid_spec,
        out_shape=jax.ShapeDtypeStruct((PADDED_N, D), f32),
    )(te, xs, W0, b0, W1, b1, W2, b2)


def kernel(x, Wg, bg, W0, b0, W1, b1, W2, b2):
    bf16 = jnp.bfloat16
    wg_p = jnp.zeros((D, LANES), f32).at[:, :E].set(Wg).astype(bf16)
    bg_p = jnp.full((1, LANES), -1e30, f32).at[0, :E].set(bg)
    idx2, rank2, counts = _route(x, wg_p, bg_p)
    slot2, te2 = _slots(counts, idx2, rank2)
    slot = slot2.reshape(N)
    te = te2.reshape(LANES)
    xs = _permute(x, slot)
    ys = _moe(te, xs.astype(bf16), W0.astype(bf16), b0.reshape(E, 1, D),
              W1.astype(bf16), b1.reshape(E, 1, D),
              W2.astype(bf16), b2.reshape(E, 1, D))
    return _unpermute(ys, slot)


# speedup vs baseline: 2.1274x; 1.0565x over previous
import functools

import jax
import jax.numpy as jnp
from jax import lax
from jax.experimental import pallas as pl
from jax.experimental.pallas import tpu as pltpu
from jax.experimental.pallas import tpu_sc as plsc

N = 8192
D = 1024
E = 8
LANES = 128
BN = 512
T = 256
NT = N // T + E
PADDED_N = NT * T

NW = 32
RPW = N // NW
C = 32

f32 = jnp.float32
i32 = jnp.int32


def _route_body(x_ref, wg_ref, bg_ref, idx_ref, rank_ref, counts_ref):
    i = pl.program_id(0)
    logits = jnp.dot(x_ref[...].astype(jnp.bfloat16), wg_ref[...],
                     preferred_element_type=f32) + bg_ref[...]
    m = jnp.max(logits, axis=1, keepdims=True)
    lane = lax.broadcasted_iota(i32, logits.shape, 1)
    idxv = jnp.min(jnp.where(logits == m, lane, LANES - 1), axis=1, keepdims=True)
    oh = (lane == idxv).astype(jnp.bfloat16)
    r = lax.broadcasted_iota(i32, (BN, BN), 0)
    c = lax.broadcasted_iota(i32, (BN, BN), 1)
    ltri = (r > c).astype(jnp.bfloat16)
    csum = jnp.dot(ltri, oh, preferred_element_type=f32)
    oh = oh.astype(f32)
    carry = jnp.where(i == 0, jnp.zeros((1, LANES), f32), counts_ref[...])
    rank = jnp.sum(oh * (csum + carry), axis=1, keepdims=True)
    idx_ref[...] = idxv
    rank_ref[...] = rank.astype(i32)
    counts_ref[...] = carry + jnp.sum(oh, axis=0, keepdims=True)


def _route(x, wg_p, bg_p):
    return pl.pallas_call(
        _route_body,
        grid=(N // BN,),
        in_specs=[
            pl.BlockSpec((BN, D), lambda i: (i, 0)),
            pl.BlockSpec((D, LANES), lambda i: (0, 0)),
            pl.BlockSpec((1, LANES), lambda i: (0, 0)),
        ],
        out_specs=[
            pl.BlockSpec((BN, 1), lambda i: (i, 0)),
            pl.BlockSpec((BN, 1), lambda i: (i, 0)),
            pl.BlockSpec((1, LANES), lambda i: (0, 0)),
        ],
        out_shape=[
            jax.ShapeDtypeStruct((N, 1), i32),
            jax.ShapeDtypeStruct((N, 1), i32),
            jax.ShapeDtypeStruct((1, LANES), f32),
        ],
    )(x, wg_p, bg_p)


def _slot_body(counts_ref, idx_ref, rank_ref, slot_ref, te_ref):
    i = pl.program_id(0)
    starts, ends = [], []
    cum = i32(0)
    for e in range(E):
        cnt = counts_ref[0, e].astype(i32)
        padded = ((cnt + (T - 1)) // T) * T
        starts.append(cum)
        cum = cum + padded
        ends.append(cum)
    idxb = idx_ref[...]
    slot = rank_ref[...]
    for e in range(E):
        slot = slot + jnp.where(idxb == e, starts[e], 0)
    slot_ref[...] = slot

    @pl.when(i == 0)
    def _():
        t = lax.broadcasted_iota(i32, (1, LANES), 1) * T
        acc = jnp.zeros((1, LANES), i32)
        for e in range(E):
            acc = acc + (t >= ends[e]).astype(i32)
        te_ref[...] = jnp.minimum(acc, E - 1)


def _slots(counts, idx2, rank2):
    return pl.pallas_call(
        _slot_body,
        grid=(N // BN,),
        in_specs=[
            pl.BlockSpec(memory_space=pltpu.SMEM),
            pl.BlockSpec((BN, 1), lambda i: (i, 0)),
            pl.BlockSpec((BN, 1), lambda i: (i, 0)),
        ],
        out_specs=[
            pl.BlockSpec((BN, 1), lambda i: (i, 0)),
            pl.BlockSpec((1, LANES), lambda i: (0, 0)),
        ],
        out_shape=[
            jax.ShapeDtypeStruct((N, 1), i32),
            jax.ShapeDtypeStruct((1, LANES), i32),
        ],
    )(counts, idx2, rank2)


def _sc_mesh():
    return plsc.VectorSubcoreMesh(core_axis_name="c", subcore_axis_name="s",
                                  num_cores=2, num_subcores=16)


NBUF = 2
NCHUNK = RPW // C


def _sc_scratch():
    st = []
    for _ in range(NBUF):
        st += [pltpu.VMEM((C,), i32), pltpu.VMEM((C, D), f32),
               pltpu.SemaphoreType.DMA, pltpu.SemaphoreType.DMA]
    return st


def _permute(x, slot):
    @functools.partial(
        pl.kernel,
        out_type=jax.ShapeDtypeStruct((PADDED_N, D), f32),
        mesh=_sc_mesh(),
        scratch_types=_sc_scratch(),
    )
    def body(x_hbm, slot_hbm, xs_hbm, *bufs):
        wid = lax.axis_index("s") * 2 + lax.axis_index("c")
        base = wid * RPW
        grp = [bufs[4 * b:4 * b + 4] for b in range(NBUF)]

        def load(j, b):
            idx_v, rows_v, sem_in, _ = grp[b]
            off = base + j * C
            pltpu.sync_copy(slot_hbm.at[pl.ds(off, C)], idx_v)
            return pltpu.async_copy(x_hbm.at[pl.ds(off, C)], rows_v, sem_in)

        pending = load(0, 0)
        for j in range(NCHUNK):
            b = j % NBUF
            idx_v, rows_v, _, sem_out = grp[b]
            pending.wait()
            if j + 1 < NCHUNK:
                pending = load(j + 1, (j + 1) % NBUF)
            pltpu.async_copy(rows_v, xs_hbm.at[idx_v], sem_out).wait()

    return body(x, slot)


def _unpermute(ys, slot):
    @functools.partial(
        pl.kernel,
        out_type=jax.ShapeDtypeStruct((N, D), f32),
        mesh=_sc_mesh(),
        scratch_types=_sc_scratch(),
    )
    def body(ys_hbm, slot_hbm, out_hbm, *bufs):
        wid = lax.axis_index("s") * 2 + lax.axis_index("c")
        base = wid * RPW
        grp = [bufs[4 * b:4 * b + 4] for b in range(NBUF)]

        def gather(j, b):
            idx_v, rows_v, sem_in, _ = grp[b]
            off = base + j * C
            pltpu.sync_copy(slot_hbm.at[pl.ds(off, C)], idx_v)
            return pltpu.async_copy(ys_hbm.at[idx_v], rows_v, sem_in)

        pending = gather(0, 0)
        for j in range(NCHUNK):
            b = j % NBUF
            _, rows_v, _, _ = grp[b]
            pending.wait()
            if j + 1 < NCHUNK:
                pending = gather(j + 1, (j + 1) % NBUF)
            pltpu.sync_copy(rows_v, out_hbm.at[pl.ds(base + j * C, C)])

    return body(ys, slot)


def _moe_body(te_ref, x_ref, w0_ref, b0_ref, w1_ref, b1_ref, w2_ref, b2_ref, y_ref):
    h = jnp.dot(x_ref[...].astype(jnp.bfloat16), w0_ref[0],
                preferred_element_type=f32)
    h = jnp.maximum(h + b0_ref[0], 0.0).astype(jnp.bfloat16)
    h = jnp.dot(h, w1_ref[0], preferred_element_type=f32)
    h = jnp.maximum(h + b1_ref[0], 0.0).astype(jnp.bfloat16)
    y_ref[...] = jnp.dot(h, w2_ref[0], preferred_element_type=f32) + b2_ref[0]


def _moe(te, xs, W0, b0, W1, b1, W2, b2):
    grid_spec = pltpu.PrefetchScalarGridSpec(
        num_scalar_prefetch=1,
        grid=(NT,),
        in_specs=[
            pl.BlockSpec((T, D), lambda i, te: (i, 0)),
            pl.BlockSpec((1, D, D), lambda i, te: (te[i], 0, 0)),
            pl.BlockSpec((1, 1, D), lambda i, te: (te[i], 0, 0)),
            pl.BlockSpec((1, D, D), lambda i, te: (te[i], 0, 0)),
            pl.BlockSpec((1, 1, D), lambda i, te: (te[i], 0, 0)),
            pl.BlockSpec((1, D, D), lambda i, te: (te[i], 0, 0)),
            pl.BlockSpec((1, 1, D), lambda i, te: (te[i], 0, 0)),
        ],
        out_specs=pl.BlockSpec((T, D), lambda i, te: (i, 0)),
    )
    return pl.pallas_call(
        _moe_body,
        grid_spec=grid_spec,
        out_shape=jax.ShapeDtypeStruct((PADDED_N, D), f32),
    )(te, xs, W0, b0, W1, b1, W2, b2)


def kernel(x, Wg, bg, W0, b0, W1, b1, W2, b2):
    bf16 = jnp.bfloat16
    wg_p = jnp.zeros((D, LANES), f32).at[:, :E].set(Wg).astype(bf16)
    bg_p = jnp.full((1, LANES), -1e30, f32).at[0, :E].set(bg)
    idx2, rank2, counts = _route(x, wg_p, bg_p)
    slot2, te2 = _slots(counts, idx2, rank2)
    slot = slot2.reshape(N)
    te = te2.reshape(LANES)
    xs = _permute(x, slot)
    ys = _moe(te, xs, W0.astype(bf16), b0.reshape(E, 1, D),
              W1.astype(bf16), b1.reshape(E, 1, D),
              W2.astype(bf16), b2.reshape(E, 1, D))
    return _unpermute(ys, slot)


# speedup vs baseline: 2.3578x; 1.1083x over previous
import functools

import jax
import jax.numpy as jnp
from jax import lax
from jax.experimental import pallas as pl
from jax.experimental.pallas import tpu as pltpu
from jax.experimental.pallas import tpu_sc as plsc

N = 8192
D = 1024
E = 8
LANES = 128
BN = 512
T = 256
NT = N // T + E
PADDED_N = NT * T

NW = 32
RPW = N // NW
C = 32

f32 = jnp.float32
i32 = jnp.int32


def _route_body(x_ref, wg_ref, bg_ref, idx_ref, rank_ref, counts_ref):
    i = pl.program_id(0)
    logits = jnp.dot(x_ref[...], wg_ref[...],
                     preferred_element_type=f32) + bg_ref[...]
    m = jnp.max(logits, axis=1, keepdims=True)
    lane = lax.broadcasted_iota(i32, logits.shape, 1)
    idxv = jnp.min(jnp.where(logits == m, lane, LANES - 1), axis=1, keepdims=True)
    oh = (lane == idxv).astype(f32)
    r = lax.broadcasted_iota(i32, (BN, BN), 0)
    c = lax.broadcasted_iota(i32, (BN, BN), 1)
    ltri = (r > c).astype(f32)
    csum = jnp.dot(ltri, oh, preferred_element_type=f32)
    carry = jnp.where(i == 0, jnp.zeros((1, LANES), f32), counts_ref[...])
    rank = jnp.sum(oh * (csum + carry), axis=1, keepdims=True)
    idx_ref[...] = idxv
    rank_ref[...] = rank.astype(i32)
    counts_ref[...] = carry + jnp.sum(oh, axis=0, keepdims=True)


def _route(x, wg_p, bg_p):
    return pl.pallas_call(
        _route_body,
        grid=(N // BN,),
        in_specs=[
            pl.BlockSpec((BN, D), lambda i: (i, 0)),
            pl.BlockSpec((D, LANES), lambda i: (0, 0)),
            pl.BlockSpec((1, LANES), lambda i: (0, 0)),
        ],
        out_specs=[
            pl.BlockSpec((BN, 1), lambda i: (i, 0)),
            pl.BlockSpec((BN, 1), lambda i: (i, 0)),
            pl.BlockSpec((1, LANES), lambda i: (0, 0)),
        ],
        out_shape=[
            jax.ShapeDtypeStruct((N, 1), i32),
            jax.ShapeDtypeStruct((N, 1), i32),
            jax.ShapeDtypeStruct((1, LANES), f32),
        ],
    )(x, wg_p, bg_p)


def _slot_body(counts_ref, idx_ref, rank_ref, slot_ref, te_ref):
    i = pl.program_id(0)
    starts, ends = [], []
    cum = i32(0)
    for e in range(E):
        cnt = counts_ref[0, e].astype(i32)
        padded = ((cnt + (T - 1)) // T) * T
        starts.append(cum)
        cum = cum + padded
        ends.append(cum)
    idxb = idx_ref[...]
    slot = rank_ref[...]
    for e in range(E):
        slot = slot + jnp.where(idxb == e, starts[e], 0)
    slot_ref[...] = slot

    @pl.when(i == 0)
    def _():
        t = lax.broadcasted_iota(i32, (1, LANES), 1) * T
        acc = jnp.zeros((1, LANES), i32)
        for e in range(E):
            acc = acc + (t >= ends[e]).astype(i32)
        te_ref[...] = jnp.minimum(acc, E - 1)


def _slots(counts, idx2, rank2):
    return pl.pallas_call(
        _slot_body,
        grid=(N // BN,),
        in_specs=[
            pl.BlockSpec(memory_space=pltpu.SMEM),
            pl.BlockSpec((BN, 1), lambda i: (i, 0)),
            pl.BlockSpec((BN, 1), lambda i: (i, 0)),
        ],
        out_specs=[
            pl.BlockSpec((BN, 1), lambda i: (i, 0)),
            pl.BlockSpec((1, LANES), lambda i: (0, 0)),
        ],
        out_shape=[
            jax.ShapeDtypeStruct((N, 1), i32),
            jax.ShapeDtypeStruct((1, LANES), i32),
        ],
    )(counts, idx2, rank2)


def _sc_mesh():
    return plsc.VectorSubcoreMesh(core_axis_name="c", subcore_axis_name="s",
                                  num_cores=2, num_subcores=16)


NBUF = 2
NCHUNK = RPW // C


def _sc_scratch():
    st = []
    for _ in range(NBUF):
        st += [pltpu.VMEM((C,), i32), pltpu.VMEM((C, D), f32),
               pltpu.SemaphoreType.DMA, pltpu.SemaphoreType.DMA]
    return st


def _permute(x, slot):
    @functools.partial(
        pl.kernel,
        out_type=jax.ShapeDtypeStruct((PADDED_N, D), f32),
        mesh=_sc_mesh(),
        scratch_types=_sc_scratch(),
    )
    def body(x_hbm, slot_hbm, xs_hbm, *bufs):
        wid = lax.axis_index("s") * 2 + lax.axis_index("c")
        base = wid * RPW
        grp = [bufs[4 * b:4 * b + 4] for b in range(NBUF)]

        def load(j, b):
            idx_v, rows_v, sem_in, _ = grp[b]
            off = base + j * C
            pltpu.sync_copy(slot_hbm.at[pl.ds(off, C)], idx_v)
            return pltpu.async_copy(x_hbm.at[pl.ds(off, C)], rows_v, sem_in)

        pending = load(0, 0)
        for j in range(NCHUNK):
            b = j % NBUF
            idx_v, rows_v, _, sem_out = grp[b]
            pending.wait()
            if j + 1 < NCHUNK:
                pending = load(j + 1, (j + 1) % NBUF)
            pltpu.async_copy(rows_v, xs_hbm.at[idx_v], sem_out).wait()

    return body(x, slot)


def _unpermute(ys, slot):
    @functools.partial(
        pl.kernel,
        out_type=jax.ShapeDtypeStruct((N, D), f32),
        mesh=_sc_mesh(),
        scratch_types=_sc_scratch(),
    )
    def body(ys_hbm, slot_hbm, out_hbm, *bufs):
        wid = lax.axis_index("s") * 2 + lax.axis_index("c")
        base = wid * RPW
        grp = [bufs[4 * b:4 * b + 4] for b in range(NBUF)]

        def gather(j, b):
            idx_v, rows_v, sem_in, _ = grp[b]
            off = base + j * C
            pltpu.sync_copy(slot_hbm.at[pl.ds(off, C)], idx_v)
            return pltpu.async_copy(ys_hbm.at[idx_v], rows_v, sem_in)

        pending = gather(0, 0)
        for j in range(NCHUNK):
            b = j % NBUF
            _, rows_v, _, _ = grp[b]
            pending.wait()
            if j + 1 < NCHUNK:
                pending = gather(j + 1, (j + 1) % NBUF)
            pltpu.sync_copy(rows_v, out_hbm.at[pl.ds(base + j * C, C)])

    return body(ys, slot)


def _moe_body(te_ref, x_ref, w0_ref, b0_ref, w1_ref, b1_ref, w2_ref, b2_ref, y_ref):
    h = jnp.dot(x_ref[...], w0_ref[0], preferred_element_type=f32)
    h = jnp.maximum(h + b0_ref[0], 0.0)
    h = jnp.dot(h, w1_ref[0], preferred_element_type=f32)
    h = jnp.maximum(h + b1_ref[0], 0.0)
    y_ref[...] = jnp.dot(h, w2_ref[0], preferred_element_type=f32) + b2_ref[0]


def _moe(te, xs, W0, b0, W1, b1, W2, b2):
    grid_spec = pltpu.PrefetchScalarGridSpec(
        num_scalar_prefetch=1,
        grid=(NT,),
        in_specs=[
            pl.BlockSpec((T, D), lambda i, te: (i, 0)),
            pl.BlockSpec((1, D, D), lambda i, te: (te[i], 0, 0)),
            pl.BlockSpec((1, 1, D), lambda i, te: (te[i], 0, 0)),
            pl.BlockSpec((1, D, D), lambda i, te: (te[i], 0, 0)),
            pl.BlockSpec((1, 1, D), lambda i, te: (te[i], 0, 0)),
            pl.BlockSpec((1, D, D), lambda i, te: (te[i], 0, 0)),
            pl.BlockSpec((1, 1, D), lambda i, te: (te[i], 0, 0)),
        ],
        out_specs=pl.BlockSpec((T, D), lambda i, te: (i, 0)),
    )
    return pl.pallas_call(
        _moe_body,
        grid_spec=grid_spec,
        out_shape=jax.ShapeDtypeStruct((PADDED_N, D), f32),
    )(te, xs, W0, b0, W1, b1, W2, b2)


def kernel(x, Wg, bg, W0, b0, W1, b1, W2, b2):
    wg_p = jnp.zeros((D, LANES), f32).at[:, :E].set(Wg)
    bg_p = jnp.full((1, LANES), -1e30, f32).at[0, :E].set(bg)
    idx2, rank2, counts = _route(x, wg_p, bg_p)
    slot2, te2 = _slots(counts, idx2, rank2)
    slot = slot2.reshape(N)
    te = te2.reshape(LANES)
    xs = _permute(x, slot)
    ys = _moe(te, xs, W0, b0.reshape(E, 1, D), W1, b1.reshape(E, 1, D),
              W2, b2.reshape(E, 1, D))
    return _unpermute(ys, slot)


# speedup vs baseline: 2.5292x; 1.0727x over previous
import functools

import jax
import jax.numpy as jnp
from jax import lax
from jax.experimental import pallas as pl
from jax.experimental.pallas import tpu as pltpu
from jax.experimental.pallas import tpu_sc as plsc

N = 8192
D = 1024
E = 8
LANES = 128
BN = 512
T = 256
NT = N // T + E
PADDED_N = NT * T

NW = 32
RPW = N // NW
C = 32

f32 = jnp.float32
i32 = jnp.int32


def _route_body(x_ref, wg_ref, bg_ref, slot_ref, te_ref, idx_s, rank_s, cnt_s):
    p = pl.program_id(0)
    i = pl.program_id(1)

    @pl.when(p == 0)
    def _phase0():
        logits = jnp.dot(x_ref[...], wg_ref[...],
                         preferred_element_type=f32) + bg_ref[...]
        m = jnp.max(logits, axis=1, keepdims=True)
        lane = lax.broadcasted_iota(i32, logits.shape, 1)
        idxv = jnp.min(jnp.where(logits == m, lane, LANES - 1), axis=1,
                       keepdims=True)
        oh = (lane == idxv).astype(f32)
        r = lax.broadcasted_iota(i32, (BN, BN), 0)
        c = lax.broadcasted_iota(i32, (BN, BN), 1)
        ltri = (r > c).astype(f32)
        csum = jnp.dot(ltri, oh, preferred_element_type=f32)
        carry = jnp.where(i == 0, jnp.zeros((1, LANES), f32), cnt_s[...])
        rank = jnp.sum(oh * (csum + carry), axis=1, keepdims=True)
        idx_s[pl.ds(i * BN, BN), :] = idxv
        rank_s[pl.ds(i * BN, BN), :] = rank.astype(i32)
        cnt_s[...] = carry + jnp.sum(oh, axis=0, keepdims=True)

    @pl.when(p == 1)
    def _phase1():
        starts, ends = [], []
        cum = i32(0)
        for e in range(E):
            cnt = cnt_s[0, e].astype(i32)
            padded = ((cnt + (T - 1)) // T) * T
            starts.append(cum)
            cum = cum + padded
            ends.append(cum)
        idxb = idx_s[pl.ds(i * BN, BN), :]
        slot = rank_s[pl.ds(i * BN, BN), :]
        for e in range(E):
            slot = slot + jnp.where(idxb == e, starts[e], 0)
        slot_ref[...] = slot

        @pl.when(i == 0)
        def _():
            t = lax.broadcasted_iota(i32, (1, LANES), 1) * T
            acc = jnp.zeros((1, LANES), i32)
            for e in range(E):
                acc = acc + (t >= ends[e]).astype(i32)
            te_ref[...] = jnp.minimum(acc, E - 1)


def _route(x, wg_p, bg_p):
    return pl.pallas_call(
        _route_body,
        grid=(2, N // BN),
        in_specs=[
            pl.BlockSpec((BN, D), lambda p, i: ((1 - p) * i, 0)),
            pl.BlockSpec((D, LANES), lambda p, i: (0, 0)),
            pl.BlockSpec((1, LANES), lambda p, i: (0, 0)),
        ],
        out_specs=[
            pl.BlockSpec((BN, 1), lambda p, i: (i, 0)),
            pl.BlockSpec((1, LANES), lambda p, i: (0, 0)),
        ],
        out_shape=[
            jax.ShapeDtypeStruct((N, 1), i32),
            jax.ShapeDtypeStruct((1, LANES), i32),
        ],
        scratch_shapes=[
            pltpu.VMEM((N, 1), i32),
            pltpu.VMEM((N, 1), i32),
            pltpu.VMEM((1, LANES), f32),
        ],
    )(x, wg_p, bg_p)


def _sc_mesh():
    return plsc.VectorSubcoreMesh(core_axis_name="c", subcore_axis_name="s",
                                  num_cores=2, num_subcores=16)


NBUF = 3
NCHUNK = RPW // C


def _sc_scratch():
    st = []
    for _ in range(NBUF):
        st += [pltpu.VMEM((C,), i32), pltpu.VMEM((C, D), f32),
               pltpu.SemaphoreType.DMA, pltpu.SemaphoreType.DMA]
    return st


def _permute(x, slot):
    @functools.partial(
        pl.kernel,
        out_type=jax.ShapeDtypeStruct((PADDED_N, D), f32),
        mesh=_sc_mesh(),
        scratch_types=_sc_scratch(),
    )
    def body(x_hbm, slot_hbm, xs_hbm, *bufs):
        wid = lax.axis_index("s") * 2 + lax.axis_index("c")
        base = wid * RPW
        grp = [bufs[4 * b:4 * b + 4] for b in range(NBUF)]
        loads = [None] * NBUF
        scats = [None] * NBUF

        def load(j):
            b = j % NBUF
            idx_v, rows_v, sem_in, _ = grp[b]
            off = base + j * C
            pltpu.sync_copy(slot_hbm.at[pl.ds(off, C)], idx_v)
            loads[b] = pltpu.async_copy(x_hbm.at[pl.ds(off, C)], rows_v, sem_in)

        load(0)
        if NCHUNK > 1:
            load(1)
        for j in range(NCHUNK):
            b = j % NBUF
            idx_v, rows_v, _, sem_out = grp[b]
            loads[b].wait()
            scats[b] = pltpu.async_copy(rows_v, xs_hbm.at[idx_v], sem_out)
            nj = j + 2
            if nj < NCHUNK:
                nb = nj % NBUF
                if scats[nb] is not None:
                    scats[nb].wait()
                    scats[nb] = None
                load(nj)
        for b in range(NBUF):
            if scats[b] is not None:
                scats[b].wait()

    return body(x, slot)


def _unpermute(ys, slot):
    @functools.partial(
        pl.kernel,
        out_type=jax.ShapeDtypeStruct((N, D), f32),
        mesh=_sc_mesh(),
        scratch_types=_sc_scratch(),
    )
    def body(ys_hbm, slot_hbm, out_hbm, *bufs):
        wid = lax.axis_index("s") * 2 + lax.axis_index("c")
        base = wid * RPW
        grp = [bufs[4 * b:4 * b + 4] for b in range(NBUF)]
        gaths = [None] * NBUF
        stores = [None] * NBUF

        def gather(j):
            b = j % NBUF
            idx_v, rows_v, sem_in, _ = grp[b]
            off = base + j * C
            pltpu.sync_copy(slot_hbm.at[pl.ds(off, C)], idx_v)
            gaths[b] = pltpu.async_copy(ys_hbm.at[idx_v], rows_v, sem_in)

        gather(0)
        if NCHUNK > 1:
            gather(1)
        for j in range(NCHUNK):
            b = j % NBUF
            _, rows_v, _, sem_out = grp[b]
            gaths[b].wait()
            stores[b] = pltpu.async_copy(rows_v, out_hbm.at[pl.ds(base + j * C, C)],
                                         sem_out)
            nj = j + 2
            if nj < NCHUNK:
                nb = nj % NBUF
                if stores[nb] is not None:
                    stores[nb].wait()
                    stores[nb] = None
                gather(nj)
        for b in range(NBUF):
            if stores[b] is not None:
                stores[b].wait()

    return body(ys, slot)


def _moe_body(te_ref, x_ref, w0_ref, b0_ref, w1_ref, b1_ref, w2_ref, b2_ref, y_ref):
    h = jnp.dot(x_ref[...], w0_ref[0], preferred_element_type=f32)
    h = jnp.maximum(h + b0_ref[0], 0.0)
    h = jnp.dot(h, w1_ref[0], preferred_element_type=f32)
    h = jnp.maximum(h + b1_ref[0], 0.0)
    y_ref[...] = jnp.dot(h, w2_ref[0], preferred_element_type=f32) + b2_ref[0]


def _moe(te, xs, W0, b0, W1, b1, W2, b2):
    grid_spec = pltpu.PrefetchScalarGridSpec(
        num_scalar_prefetch=1,
        grid=(NT,),
        in_specs=[
            pl.BlockSpec((T, D), lambda i, te: (i, 0)),
            pl.BlockSpec((1, D, D), lambda i, te: (te[i], 0, 0)),
            pl.BlockSpec((1, 1, D), lambda i, te: (te[i], 0, 0)),
            pl.BlockSpec((1, D, D), lambda i, te: (te[i], 0, 0)),
            pl.BlockSpec((1, 1, D), lambda i, te: (te[i], 0, 0)),
            pl.BlockSpec((1, D, D), lambda i, te: (te[i], 0, 0)),
            pl.BlockSpec((1, 1, D), lambda i, te: (te[i], 0, 0)),
        ],
        out_specs=pl.BlockSpec((T, D), lambda i, te: (i, 0)),
    )
    return pl.pallas_call(
        _moe_body,
        grid_spec=grid_spec,
        out_shape=jax.ShapeDtypeStruct((PADDED_N, D), f32),
    )(te, xs, W0, b0, W1, b1, W2, b2)


def kernel(x, Wg, bg, W0, b0, W1, b1, W2, b2):
    wg_p = jnp.zeros((D, LANES), f32).at[:, :E].set(Wg)
    bg_p = jnp.full((1, LANES), -1e30, f32).at[0, :E].set(bg)
    slot2, te2 = _route(x, wg_p, bg_p)
    slot = slot2.reshape(N)
    te = te2.reshape(LANES)
    xs = _permute(x, slot)
    ys = _moe(te, xs, W0, b0.reshape(E, 1, D), W1, b1.reshape(E, 1, D),
              W2, b2.reshape(E, 1, D))
    return _unpermute(ys, slot)
